# baseline (speedup 1.0000x reference)
import functools

import jax
import jax.numpy as jnp
from jax import lax
from jax.experimental import pallas as pl
from jax.experimental.pallas import tpu as pltpu
from jax.experimental.pallas import tpu_sc as plsc

_R = 16384
_S = 50
_SP = 64
_D = 32
_SD = _SP * _D

_info = plsc.get_sparse_core_info()
_NC, _NS = _info.num_cores, _info.num_subcores
_NW = _NC * _NS
_RPW = _R // _NW
_RCH = 32
_NCHUNK = _RPW // _RCH

_mesh = plsc.VectorSubcoreMesh(core_axis_name="c", subcore_axis_name="s")


@functools.partial(
    pl.kernel,
    mesh=_mesh,
    compiler_params=pltpu.CompilerParams(use_tc_tiling_on_sc=False),
    out_type=jax.ShapeDtypeStruct((_R, _SP, _D), jnp.float32),
    scratch_types=[
        pltpu.VMEM((_RCH * _S,), jnp.int32),
        pltpu.VMEM((_RCH * _S, _D), jnp.float32),
        pltpu.SemaphoreType.DMA,
        pltpu.SemaphoreType.DMA,
    ],
)
def _gather_rows(idx_hbm, table_hbm, out_hbm, idx_v, rows_v, gsem, osem):
    wid = lax.axis_index("s") * _NC + lax.axis_index("c")
    base = wid * _RPW

    def body(i, carry):
        r0 = base + i * _RCH
        pltpu.sync_copy(idx_hbm.at[pl.ds(r0 * _S, _RCH * _S)], idx_v)
        pltpu.async_copy(table_hbm.at[idx_v], rows_v, gsem).wait()
        cps = [
            pltpu.async_copy(rows_v.at[pl.ds(j * _S, _S)],
                             out_hbm.at[r0 + j, pl.ds(0, _S)], osem)
            for j in range(_RCH)
        ]
        for cp in cps:
            cp.wait()
        return carry

    lax.fori_loop(0, _NCHUNK, body, 0)


_TBLK = 512


def _transpose_body(x_ref, o_ref):
    o_ref[...] = jnp.transpose(x_ref[...], (1, 0))


_transpose = pl.pallas_call(
    _transpose_body,
    grid=(_R // _TBLK,),
    in_specs=[pl.BlockSpec((_TBLK, _SD), lambda i: (i, 0))],
    out_specs=pl.BlockSpec((_SD, _TBLK), lambda i: (0, i)),
    out_shape=jax.ShapeDtypeStruct((_SD, _R), jnp.float32),
)


def kernel(band_id, table):
    idx = band_id.reshape(-1).astype(jnp.int32)
    rows = _gather_rows(idx, table)
    t = _transpose(rows.reshape(_R, _SD))
    return jnp.transpose(t.reshape(_SP, _D, _R)[:_S], (2, 0, 1))


# speedup vs baseline: 1.5755x; 1.5755x over previous
import functools

import jax
import jax.numpy as jnp
from jax import lax
from jax.experimental import pallas as pl
from jax.experimental.pallas import tpu as pltpu
from jax.experimental.pallas import tpu_sc as plsc

_R = 16384
_S = 50
_D = 32

_info = plsc.get_sparse_core_info()
_NC, _NS = _info.num_cores, _info.num_subcores
_NW = _NC * _NS
_RPW = _R // _NW
_RCH = 32
_NCHUNK = _RPW // _RCH

_mesh = plsc.VectorSubcoreMesh(core_axis_name="c", subcore_axis_name="s")


@functools.partial(
    pl.kernel,
    mesh=_mesh,
    compiler_params=pltpu.CompilerParams(use_tc_tiling_on_sc=False),
    out_type=jax.ShapeDtypeStruct((_R, _S, _D), jnp.float32),
    scratch_types=[
        pltpu.VMEM((_RCH * _S,), jnp.int32),
        pltpu.VMEM((_RCH * _S, _D), jnp.float32),
        pltpu.SemaphoreType.DMA,
        pltpu.SemaphoreType.DMA,
    ],
)
def _gather_rows(idx_hbm, table_hbm, out_hbm, idx_v, rows_v, gsem, osem):
    wid = lax.axis_index("s") * _NC + lax.axis_index("c")
    base = wid * _RPW

    def body(i, carry):
        r0 = base + i * _RCH
        pltpu.sync_copy(idx_hbm.at[pl.ds(r0 * _S, _RCH * _S)], idx_v)
        pltpu.async_copy(table_hbm.at[idx_v], rows_v, gsem).wait()
        cps = [
            pltpu.async_copy(rows_v.at[pl.ds(j * _S, _S)],
                             out_hbm.at[r0 + j], osem)
            for j in range(_RCH)
        ]
        for cp in cps:
            cp.wait()
        return carry

    lax.fori_loop(0, _NCHUNK, body, 0)


def kernel(band_id, table):
    idx = band_id.reshape(-1).astype(jnp.int32)
    return _gather_rows(idx, table)
